# Initial kernel scaffold; baseline (speedup 1.0000x reference)
import jax
import jax.numpy as jnp
from jax.experimental import pallas as pl


def kernel(x, z, edge_index, kernel_offset, W1, gamma1, beta1, W2, Wp, bp, gamma_p, beta_p):
    raise NotImplementedError("write your pallas kernel here")


# speedup vs baseline: 2.1902x; 2.1902x over previous
import functools

import jax
import jax.numpy as jnp
from jax import lax
from jax.experimental import pallas as pl
from jax.experimental.pallas import tpu as pltpu
from jax.experimental.pallas import tpu_sc as plsc

_N = 10000
_E = 320000
_F = 128
_K = 27

_NC = 2
_NS = 16
_NW = _NC * _NS
_C = 128
_CHUNKS = 79
_PER_W = _C * _CHUNKS
_E_PAD = _PER_W * _NW
_ROWS_PAD = 10240
_ZROWS = 640
_DUMMY_ROW = 10008


def _sc_gather_scatter(y_flat, src_flat, dst):
    mesh = plsc.VectorSubcoreMesh(core_axis_name="c", subcore_axis_name="s")

    @functools.partial(
        pl.kernel,
        mesh=mesh,
        out_type=jax.ShapeDtypeStruct((_NC, _ROWS_PAD, _F), jnp.float32),
        scratch_types=[
            pltpu.VMEM((_C,), jnp.int32),
            pltpu.VMEM((_C,), jnp.int32),
            pltpu.VMEM((_C, _F), jnp.float32),
            pltpu.VMEM((_C, _F), jnp.float32),
            pltpu.VMEM_SHARED((_ROWS_PAD, _F), jnp.float32),
            pltpu.SemaphoreType.DMA,
        ],
    )
    def k(y_hbm, src_hbm, dst_hbm, out_hbm, src_v, dst_v, rows_v, zero_v, acc, sem):
        cid = lax.axis_index("c")
        sid = lax.axis_index("s")
        wid = sid * _NC + cid

        def zrow(i, carry):
            for j in range(_F // 16):
                zero_v[i, pl.ds(j * 16, 16)] = jnp.zeros((16,), jnp.float32)
            return carry

        lax.fori_loop(0, _C, zrow, 0)

        base_z = sid * _ZROWS
        for j in range(_ZROWS // _C):
            pltpu.sync_copy(zero_v, acc.at[pl.ds(base_z + j * _C, _C)])
        plsc.subcore_barrier()

        base_e = wid * _PER_W

        def step(i, carry):
            off = pl.multiple_of(base_e + i * _C, _C)
            pltpu.sync_copy(src_hbm.at[pl.ds(off, _C)], src_v)
            pltpu.sync_copy(dst_hbm.at[pl.ds(off, _C)], dst_v)
            pltpu.async_copy(y_hbm.at[src_v], rows_v, sem).wait()
            pltpu.sync_copy(rows_v, acc.at[dst_v], add=True)
            return carry

        lax.fori_loop(0, _CHUNKS, step, 0)
        plsc.subcore_barrier()

        pltpu.sync_copy(acc.at[pl.ds(base_z, _ZROWS)],
                        out_hbm.at[cid, pl.ds(base_z, _ZROWS)])

    return k(y_flat, src_flat, dst)


def _tc_einsum(feat, W):
    def body(f_ref, w_ref, y_ref):
        y_ref[0] = jnp.dot(f_ref[...], w_ref[0], preferred_element_type=jnp.float32)

    return pl.pallas_call(
        body,
        grid=(_K,),
        in_specs=[
            pl.BlockSpec((_N, _F), lambda k: (0, 0)),
            pl.BlockSpec((1, _F, _F), lambda k: (k, 0, 0)),
        ],
        out_specs=pl.BlockSpec((1, _N, _F), lambda k: (k, 0, 0)),
        out_shape=jax.ShapeDtypeStruct((_K, _N, _F), jnp.float32),
    )(feat, W)


def _tc_bn_silu(p, gamma, beta):
    def body(p_ref, g_ref, b_ref, o_ref):
        h = p_ref[0, :_N] + p_ref[1, :_N]
        mu = jnp.mean(h, axis=0, keepdims=True)
        var = jnp.mean(jnp.square(h - mu), axis=0, keepdims=True)
        hn = (h - mu) * lax.rsqrt(var + 1e-5) * g_ref[...] + b_ref[...]
        o_ref[...] = hn * jax.nn.sigmoid(hn)

    return pl.pallas_call(
        body,
        out_shape=jax.ShapeDtypeStruct((_N, _F), jnp.float32),
    )(p, gamma.reshape(1, _F), beta.reshape(1, _F))


def _tc_final(p, z, Wp, bp, gamma_p, beta_p):
    def body(p_ref, z_ref, w_ref, bp_ref, g_ref, b_ref, o_ref):
        h = p_ref[0, :_N] + p_ref[1, :_N]
        zp = jnp.dot(z_ref[...], w_ref[...], preferred_element_type=jnp.float32)
        zp = zp + bp_ref[...]
        mu = jnp.mean(zp, axis=0, keepdims=True)
        var = jnp.mean(jnp.square(zp - mu), axis=0, keepdims=True)
        zpn = (zp - mu) * lax.rsqrt(var + 1e-5) * g_ref[...] + b_ref[...]
        zpn = jnp.maximum(zpn, 0.0)
        o_ref[...] = h + zpn

    return pl.pallas_call(
        body,
        out_shape=jax.ShapeDtypeStruct((_N, _F), jnp.float32),
    )(p, z, Wp, bp.reshape(1, _F), gamma_p.reshape(1, _F),
      beta_p.reshape(1, _F))


def kernel(x, z, edge_index, kernel_offset, W1, gamma1, beta1, W2, Wp, bp, gamma_p, beta_p):
    src = edge_index[0]
    dst = edge_index[1]
    flat_src = kernel_offset * _N + src
    pad = _E_PAD - _E
    flat_src = jnp.concatenate([flat_src, jnp.zeros((pad,), jnp.int32)])
    dst_p = jnp.concatenate([dst, jnp.full((pad,), _DUMMY_ROW, jnp.int32)])

    y1 = _tc_einsum(x, W1).reshape(_K * _N, _F)
    p1 = _sc_gather_scatter(y1, flat_src, dst_p)
    h1 = _tc_bn_silu(p1, gamma1, beta1)
    y2 = _tc_einsum(h1, W2).reshape(_K * _N, _F)
    p2 = _sc_gather_scatter(y2, flat_src, dst_p)
    out = _tc_final(p2, z, Wp, bp, gamma_p, beta_p)
    return (out, out)
